# baseline (speedup 1.0000x reference)
import jax
import jax.numpy as jnp
from jax import lax
from jax.experimental import pallas as pl
from jax.experimental.pallas import tpu as pltpu
from jax.experimental.pallas import tpu_sc as plsc

WORD_DIM = 100
PADDED_DIM = 128
LANES = 16
NUM_CORES = 2
NUM_SUBCORES = 16
NUM_WORKERS = NUM_CORES * NUM_SUBCORES

BATCH = 4096
SEQ = 200
TOTAL = BATCH * SEQ
ROWS_PER_WORKER = BATCH // NUM_WORKERS
GROUP = 8
GROUPS = ROWS_PER_WORKER // GROUP
COL_OFFS = (0, 16, 32, 48, 64, 80, 84)


def _emb_body(widx_hbm, eidx_hbm, wtab_hbm, etab_hbm, out_hbm,
              widx_v, eidx_v, buf_a, buf_b, buf_o, sem_a, sem_b):
    wid = lax.axis_index("s") * NUM_CORES + lax.axis_index("c")
    row0 = wid * ROWS_PER_WORKER

    def group_body(g, _):
        grow = row0 + g * GROUP
        pltpu.sync_copy(widx_hbm.at[pl.ds(grow, GROUP)], widx_v)
        pltpu.sync_copy(eidx_hbm.at[pl.ds(grow, GROUP)], eidx_v)

        def row_body(r, _):
            copies = [
                pltpu.async_copy(wtab_hbm.at[widx_v.at[r, pl.ds(0, 128)]],
                                 buf_a.at[pl.ds(0, 128)], sem_a),
                pltpu.async_copy(wtab_hbm.at[widx_v.at[r, pl.ds(128, 72)]],
                                 buf_a.at[pl.ds(128, 72)], sem_a),
                pltpu.async_copy(etab_hbm.at[eidx_v.at[r, pl.ds(0, 128)]],
                                 buf_b.at[pl.ds(0, 128)], sem_b),
                pltpu.async_copy(etab_hbm.at[eidx_v.at[r, pl.ds(128, 72)]],
                                 buf_b.at[pl.ds(128, 72)], sem_b),
            ]
            for c in copies:
                c.wait()

            def add_row(t, _):
                vals = [buf_a[t, pl.ds(c, LANES)] + buf_b[t, pl.ds(c, LANES)]
                        for c in COL_OFFS]
                for c, v in zip(COL_OFFS, vals):
                    buf_o[t, pl.ds(c, LANES)] = v
                return ()

            lax.fori_loop(0, SEQ, add_row, ())
            pltpu.sync_copy(buf_o, out_hbm.at[grow + r])
            return ()

        lax.fori_loop(0, GROUP, row_body, ())
        return ()

    lax.fori_loop(0, GROUPS, group_body, ())


@jax.jit
def _emb_call(widx, eidx, wtab, etab):
    mesh = plsc.VectorSubcoreMesh(core_axis_name="c", subcore_axis_name="s")
    f = pl.kernel(
        _emb_body,
        out_type=jax.ShapeDtypeStruct((BATCH, SEQ, WORD_DIM), jnp.float32),
        mesh=mesh,
        scratch_types=[
            pltpu.VMEM((GROUP, SEQ), jnp.int32),
            pltpu.VMEM((GROUP, SEQ), jnp.int32),
            pltpu.VMEM((SEQ, PADDED_DIM), jnp.float32),
            pltpu.VMEM((SEQ, PADDED_DIM), jnp.float32),
            pltpu.VMEM((SEQ, WORD_DIM), jnp.float32),
            pltpu.SemaphoreType.DMA,
            pltpu.SemaphoreType.DMA,
        ],
    )
    return f(widx, eidx, wtab, etab)


def _pad_block(in_ref, out_ref):
    out_ref[:, :WORD_DIM] = in_ref[...]


def _pad_table(tab, block_rows):
    rows = tab.shape[0]
    return pl.pallas_call(
        _pad_block,
        grid=(rows // block_rows,),
        in_specs=[pl.BlockSpec((block_rows, WORD_DIM), lambda i: (i, 0))],
        out_specs=pl.BlockSpec((block_rows, PADDED_DIM), lambda i: (i, 0)),
        out_shape=jax.ShapeDtypeStruct((rows, PADDED_DIM), jnp.float32),
    )(tab)


def kernel(words_idx, extwords_idx, word_emb, ext_emb):
    wtab = _pad_table(word_emb, 10000)
    etab = _pad_table(ext_emb, 10000)
    return _emb_call(words_idx, extwords_idx, wtab, etab)


# speedup vs baseline: 1.0474x; 1.0474x over previous
import jax
import jax.numpy as jnp
from jax import lax
from jax.experimental import pallas as pl
from jax.experimental.pallas import tpu as pltpu
from jax.experimental.pallas import tpu_sc as plsc

WORD_DIM = 100
PADDED_DIM = 128
LANES = 16
NUM_CORES = 2
NUM_SUBCORES = 16
NUM_WORKERS = NUM_CORES * NUM_SUBCORES

BATCH = 4096
SEQ = 200
TOTAL = BATCH * SEQ
ROWS_PER_WORKER = BATCH // NUM_WORKERS
GROUP = 8
GROUPS = ROWS_PER_WORKER // GROUP
COL_OFFS = (0, 16, 32, 48, 64, 80, 84)


def _emb_body(widx_hbm, eidx_hbm, wtab_hbm, etab_hbm, out_hbm,
              widx_v, eidx_v, buf_a, buf_b, buf_o, sem_a, sem_b):
    wid = lax.axis_index("s") * NUM_CORES + lax.axis_index("c")
    row0 = wid * ROWS_PER_WORKER

    def group_body(g, _):
        grow = row0 + g * GROUP
        pltpu.sync_copy(widx_hbm.at[pl.ds(grow, GROUP)], widx_v)
        pltpu.sync_copy(eidx_hbm.at[pl.ds(grow, GROUP)], eidx_v)

        def row_body(r, _):
            copies = [
                pltpu.async_copy(wtab_hbm.at[widx_v.at[r, pl.ds(0, 128)]],
                                 buf_a.at[pl.ds(0, 128)], sem_a),
                pltpu.async_copy(wtab_hbm.at[widx_v.at[r, pl.ds(128, 72)]],
                                 buf_a.at[pl.ds(128, 72)], sem_a),
                pltpu.async_copy(etab_hbm.at[eidx_v.at[r, pl.ds(0, 128)]],
                                 buf_b.at[pl.ds(0, 128)], sem_b),
                pltpu.async_copy(etab_hbm.at[eidx_v.at[r, pl.ds(128, 72)]],
                                 buf_b.at[pl.ds(128, 72)], sem_b),
            ]
            for c in copies:
                c.wait()

            def add_row(t, _):
                vals = [buf_a[t, pl.ds(c, LANES)] + buf_b[t, pl.ds(c, LANES)]
                        for c in COL_OFFS]
                for c, v in zip(COL_OFFS, vals):
                    buf_o[t, pl.ds(c, LANES)] = v
                return ()

            lax.fori_loop(0, SEQ, add_row, ())
            base = (grow + r) * SEQ
            pltpu.sync_copy(buf_o, out_hbm.at[pl.ds(base, SEQ)])
            return ()

        lax.fori_loop(0, GROUP, row_body, ())
        return ()

    lax.fori_loop(0, GROUPS, group_body, ())


@jax.jit
def _emb_call(widx, eidx, wtab, etab):
    mesh = plsc.VectorSubcoreMesh(core_axis_name="c", subcore_axis_name="s")
    f = pl.kernel(
        _emb_body,
        out_type=jax.ShapeDtypeStruct((TOTAL, WORD_DIM), jnp.float32),
        mesh=mesh,
        scratch_types=[
            pltpu.VMEM((GROUP, SEQ), jnp.int32),
            pltpu.VMEM((GROUP, SEQ), jnp.int32),
            pltpu.VMEM((SEQ, PADDED_DIM), jnp.float32),
            pltpu.VMEM((SEQ, PADDED_DIM), jnp.float32),
            pltpu.VMEM((SEQ, WORD_DIM), jnp.float32),
            pltpu.SemaphoreType.DMA,
            pltpu.SemaphoreType.DMA,
        ],
    )
    return f(widx, eidx, wtab, etab)


def _pad_block(in_ref, out_ref):
    out_ref[...] = jnp.pad(in_ref[...], ((0, 0), (0, PADDED_DIM - WORD_DIM)))


def _pad_table(tab, block_rows):
    rows = tab.shape[0]
    return pl.pallas_call(
        _pad_block,
        grid=(rows // block_rows,),
        in_specs=[pl.BlockSpec((block_rows, WORD_DIM), lambda i: (i, 0))],
        out_specs=pl.BlockSpec((block_rows, PADDED_DIM), lambda i: (i, 0)),
        out_shape=jax.ShapeDtypeStruct((rows, PADDED_DIM), jnp.float32),
    )(tab)


def kernel(words_idx, extwords_idx, word_emb, ext_emb):
    wtab = _pad_table(word_emb, 20000)
    etab = _pad_table(ext_emb, 20000)
    out = _emb_call(words_idx, extwords_idx, wtab, etab)
    return out.reshape(words_idx.shape + (WORD_DIM,))


# speedup vs baseline: 1.1753x; 1.1222x over previous
import jax
import jax.numpy as jnp
from jax import lax
from jax.experimental import pallas as pl
from jax.experimental.pallas import tpu as pltpu
from jax.experimental.pallas import tpu_sc as plsc

WORD_DIM = 100
PADDED_DIM = 128
LANES = 16
NUM_CORES = 2
NUM_SUBCORES = 16
NUM_WORKERS = NUM_CORES * NUM_SUBCORES

BATCH = 4096
SEQ = 200
TOTAL = BATCH * SEQ
ROWS_PER_WORKER = BATCH // NUM_WORKERS
GROUP = 8
BODY_ROWS = 2 * GROUP
OUTER = ROWS_PER_WORKER // BODY_ROWS
C0 = 128
C1 = SEQ - C0
COL_OFFS = (0, 16, 32, 48, 64, 80, 84)


def _emb_body(widx_hbm, eidx_hbm, wtab_hbm, etab_hbm, out_hbm,
              wiA, eiA, wiB, eiB, a0, b0, a1, b1, o0, o1,
              semA0, semB0, semA1, semB1, semIA, semIB):
    wid = lax.axis_index("s") * NUM_CORES + lax.axis_index("c")
    row0 = wid * ROWS_PER_WORKER
    stage_cap = row0 + ROWS_PER_WORKER - GROUP

    def fire0(k8, wi, ei):
        pltpu.async_copy(wtab_hbm.at[wi.at[k8, pl.ds(0, C0)]], a0, semA0)
        pltpu.async_copy(etab_hbm.at[ei.at[k8, pl.ds(0, C0)]], b0, semB0)

    def fire1(k8, wi, ei):
        pltpu.async_copy(wtab_hbm.at[wi.at[k8, pl.ds(C0, C1)]], a1, semA1)
        pltpu.async_copy(etab_hbm.at[ei.at[k8, pl.ds(C0, C1)]], b1, semB1)

    def drain0(k8, wi, ei):
        pltpu.make_async_copy(wtab_hbm.at[wi.at[k8, pl.ds(0, C0)]],
                              a0, semA0).wait()
        pltpu.make_async_copy(etab_hbm.at[ei.at[k8, pl.ds(0, C0)]],
                              b0, semB0).wait()

    def drain1(k8, wi, ei):
        pltpu.make_async_copy(wtab_hbm.at[wi.at[k8, pl.ds(C0, C1)]],
                              a1, semA1).wait()
        pltpu.make_async_copy(etab_hbm.at[ei.at[k8, pl.ds(C0, C1)]],
                              b1, semB1).wait()

    def stage(gstart, wi, ei, sem):
        pltpu.async_copy(widx_hbm.at[pl.ds(gstart, GROUP)], wi, sem)
        pltpu.async_copy(eidx_hbm.at[pl.ds(gstart, GROUP)], ei, sem)

    def drain_stage(wi, ei, sem):
        pltpu.make_async_copy(widx_hbm.at[pl.ds(row0, GROUP)], wi, sem).wait()
        pltpu.make_async_copy(eidx_hbm.at[pl.ds(row0, GROUP)], ei, sem).wait()

    def add_chunk(n, src_a, src_b, dst):
        def add_row(t, _):
            vals = [src_a[t, pl.ds(c, LANES)] + src_b[t, pl.ds(c, LANES)]
                    for c in COL_OFFS]
            for c, v in zip(COL_OFFS, vals):
                dst[t, pl.ds(c, LANES)] = v
            return ()

        lax.fori_loop(0, n, add_row, ())

    pltpu.sync_copy(widx_hbm.at[pl.ds(row0, GROUP)], wiA)
    pltpu.sync_copy(eidx_hbm.at[pl.ds(row0, GROUP)], eiA)
    stage(row0 + GROUP, wiB, eiB, semIB)
    fire0(0, wiA, eiA)

    def body(w, _):
        for k in range(BODY_ROWS):
            k8 = k % GROUP
            wi, ei = (wiA, eiA) if k < GROUP else (wiB, eiB)
            r = BODY_ROWS * w + k
            base = (row0 + r) * SEQ

            fire1(k8, wi, ei)
            drain0(k8, wi, ei)
            add_chunk(C0, a0, b0, o0)
            pltpu.sync_copy(o0, out_hbm.at[pl.ds(base, C0)])

            if k < BODY_ROWS - 1:
                nk = k + 1
                nwi, nei = (wiA, eiA) if nk < GROUP else (wiB, eiB)
                if nk == GROUP:
                    drain_stage(wiB, eiB, semIB)
                fire0(nk % GROUP, nwi, nei)
            else:
                drain_stage(wiA, eiA, semIA)
                fire0(0, wiA, eiA)
            drain1(k8, wi, ei)
            add_chunk(C1, a1, b1, o1)
            pltpu.sync_copy(o1, out_hbm.at[pl.ds(base + C0, C1)])

            if k == GROUP - 1:
                gs = jnp.minimum(row0 + BODY_ROWS * (w + 1), stage_cap)
                stage(gs, wiA, eiA, semIA)
            if k == BODY_ROWS - 1:
                gs = jnp.minimum(row0 + BODY_ROWS * (w + 1) + GROUP, stage_cap)
                stage(gs, wiB, eiB, semIB)
        return ()

    lax.fori_loop(0, OUTER, body, ())

    drain0(0, wiA, eiA)
    drain_stage(wiB, eiB, semIB)


@jax.jit
def _emb_call(widx, eidx, wtab, etab):
    mesh = plsc.VectorSubcoreMesh(core_axis_name="c", subcore_axis_name="s")
    f = pl.kernel(
        _emb_body,
        out_type=jax.ShapeDtypeStruct((TOTAL, WORD_DIM), jnp.float32),
        mesh=mesh,
        scratch_types=[
            pltpu.VMEM((GROUP, SEQ), jnp.int32),
            pltpu.VMEM((GROUP, SEQ), jnp.int32),
            pltpu.VMEM((GROUP, SEQ), jnp.int32),
            pltpu.VMEM((GROUP, SEQ), jnp.int32),
            pltpu.VMEM((C0, PADDED_DIM), jnp.float32),
            pltpu.VMEM((C0, PADDED_DIM), jnp.float32),
            pltpu.VMEM((C1, PADDED_DIM), jnp.float32),
            pltpu.VMEM((C1, PADDED_DIM), jnp.float32),
            pltpu.VMEM((C0, WORD_DIM), jnp.float32),
            pltpu.VMEM((C1, WORD_DIM), jnp.float32),
            pltpu.SemaphoreType.DMA,
            pltpu.SemaphoreType.DMA,
            pltpu.SemaphoreType.DMA,
            pltpu.SemaphoreType.DMA,
            pltpu.SemaphoreType.DMA,
            pltpu.SemaphoreType.DMA,
        ],
    )
    return f(widx, eidx, wtab, etab)


def _pad_block(in_ref, out_ref):
    out_ref[...] = jnp.pad(in_ref[...], ((0, 0), (0, PADDED_DIM - WORD_DIM)))


def _pad_table(tab, block_rows):
    rows = tab.shape[0]
    return pl.pallas_call(
        _pad_block,
        grid=(rows // block_rows,),
        in_specs=[pl.BlockSpec((block_rows, WORD_DIM), lambda i: (i, 0))],
        out_specs=pl.BlockSpec((block_rows, PADDED_DIM), lambda i: (i, 0)),
        out_shape=jax.ShapeDtypeStruct((rows, PADDED_DIM), jnp.float32),
    )(tab)


def kernel(words_idx, extwords_idx, word_emb, ext_emb):
    wtab = _pad_table(word_emb, 20000)
    etab = _pad_table(ext_emb, 20000)
    out = _emb_call(words_idx, extwords_idx, wtab, etab)
    return out.reshape(words_idx.shape + (WORD_DIM,))


# speedup vs baseline: 1.1768x; 1.0012x over previous
import jax
import jax.numpy as jnp
from jax import lax
from jax.experimental import pallas as pl
from jax.experimental.pallas import tpu as pltpu
from jax.experimental.pallas import tpu_sc as plsc

WORD_DIM = 100
PADDED_DIM = 128
LANES = 16
NUM_CORES = 2
NUM_SUBCORES = 16
NUM_WORKERS = NUM_CORES * NUM_SUBCORES

BATCH = 4096
SEQ = 200
TOTAL = BATCH * SEQ
ROWS_PER_WORKER = BATCH // NUM_WORKERS
GROUP = 8
BODY_ROWS = 2 * GROUP
OUTER = ROWS_PER_WORKER // BODY_ROWS
C0 = 128
C1 = SEQ - C0
COL_OFFS = (0, 16, 32, 48, 64, 80, 84)


def _emb_body(widx_hbm, eidx_hbm, wtab_hbm, etab_hbm, out_hbm,
              wiA, eiA, wiB, eiB, a0, b0, a1, b1, o0, o1,
              semA0, semB0, semA1, semB1, semIA, semIB):
    wid = lax.axis_index("s") * NUM_CORES + lax.axis_index("c")
    row0 = wid * ROWS_PER_WORKER
    stage_cap = row0 + ROWS_PER_WORKER - GROUP

    def fire0(k8, wi, ei):
        pltpu.async_copy(wtab_hbm.at[wi.at[k8, pl.ds(0, C0)]], a0, semA0)
        pltpu.async_copy(etab_hbm.at[ei.at[k8, pl.ds(0, C0)]], b0, semB0)

    def fire1(k8, wi, ei):
        pltpu.async_copy(wtab_hbm.at[wi.at[k8, pl.ds(C0, C1)]], a1, semA1)
        pltpu.async_copy(etab_hbm.at[ei.at[k8, pl.ds(C0, C1)]], b1, semB1)

    def drain0(k8, wi, ei):
        pltpu.make_async_copy(wtab_hbm.at[wi.at[k8, pl.ds(0, C0)]],
                              a0, semA0).wait()
        pltpu.make_async_copy(etab_hbm.at[ei.at[k8, pl.ds(0, C0)]],
                              b0, semB0).wait()

    def drain1(k8, wi, ei):
        pltpu.make_async_copy(wtab_hbm.at[wi.at[k8, pl.ds(C0, C1)]],
                              a1, semA1).wait()
        pltpu.make_async_copy(etab_hbm.at[ei.at[k8, pl.ds(C0, C1)]],
                              b1, semB1).wait()

    def stage(gstart, wi, ei, sem):
        pltpu.async_copy(widx_hbm.at[pl.ds(gstart, GROUP)], wi, sem)
        pltpu.async_copy(eidx_hbm.at[pl.ds(gstart, GROUP)], ei, sem)

    def drain_stage(wi, ei, sem):
        pltpu.make_async_copy(widx_hbm.at[pl.ds(row0, GROUP)], wi, sem).wait()
        pltpu.make_async_copy(eidx_hbm.at[pl.ds(row0, GROUP)], ei, sem).wait()

    def add_chunk(n, src_a, src_b, dst):
        def add_row(t, _):
            vals = [src_a[t, pl.ds(c, LANES)] + src_b[t, pl.ds(c, LANES)]
                    for c in COL_OFFS]
            for c, v in zip(COL_OFFS, vals):
                dst[t, pl.ds(c, LANES)] = v
            return ()

        lax.fori_loop(0, n, add_row, ())

    pltpu.sync_copy(widx_hbm.at[pl.ds(row0, GROUP)], wiA)
    pltpu.sync_copy(eidx_hbm.at[pl.ds(row0, GROUP)], eiA)
    stage(row0 + GROUP, wiB, eiB, semIB)
    fire0(0, wiA, eiA)

    def body(w, _):
        for k in range(BODY_ROWS):
            k8 = k % GROUP
            wi, ei = (wiA, eiA) if k < GROUP else (wiB, eiB)
            r = BODY_ROWS * w + k
            base = (row0 + r) * SEQ

            fire1(k8, wi, ei)
            drain0(k8, wi, ei)
            add_chunk(C0, a0, b0, o0)
            pltpu.sync_copy(o0, out_hbm.at[pl.ds(base, C0)])

            if k < BODY_ROWS - 1:
                nk = k + 1
                nwi, nei = (wiA, eiA) if nk < GROUP else (wiB, eiB)
                if nk == GROUP:
                    drain_stage(wiB, eiB, semIB)
                fire0(nk % GROUP, nwi, nei)
            else:
                drain_stage(wiA, eiA, semIA)
                fire0(0, wiA, eiA)
            drain1(k8, wi, ei)
            add_chunk(C1, a1, b1, o1)
            pltpu.sync_copy(o1, out_hbm.at[pl.ds(base + C0, C1)])

            if k == GROUP - 1:
                gs = jnp.minimum(row0 + BODY_ROWS * (w + 1), stage_cap)
                stage(gs, wiA, eiA, semIA)
            if k == BODY_ROWS - 1:
                gs = jnp.minimum(row0 + BODY_ROWS * (w + 1) + GROUP, stage_cap)
                stage(gs, wiB, eiB, semIB)
        return ()

    lax.fori_loop(0, OUTER, body, ())

    drain0(0, wiA, eiA)
    drain_stage(wiB, eiB, semIB)


@jax.jit
def _emb_call(widx, eidx, wtab, etab):
    mesh = plsc.VectorSubcoreMesh(core_axis_name="c", subcore_axis_name="s")
    f = pl.kernel(
        _emb_body,
        out_type=jax.ShapeDtypeStruct((TOTAL, WORD_DIM), jnp.float32),
        mesh=mesh,
        scratch_types=[
            pltpu.VMEM((GROUP, SEQ), jnp.int32),
            pltpu.VMEM((GROUP, SEQ), jnp.int32),
            pltpu.VMEM((GROUP, SEQ), jnp.int32),
            pltpu.VMEM((GROUP, SEQ), jnp.int32),
            pltpu.VMEM((C0, PADDED_DIM), jnp.float32),
            pltpu.VMEM((C0, PADDED_DIM), jnp.float32),
            pltpu.VMEM((C1, PADDED_DIM), jnp.float32),
            pltpu.VMEM((C1, PADDED_DIM), jnp.float32),
            pltpu.VMEM((C0, WORD_DIM), jnp.float32),
            pltpu.VMEM((C1, WORD_DIM), jnp.float32),
            pltpu.SemaphoreType.DMA,
            pltpu.SemaphoreType.DMA,
            pltpu.SemaphoreType.DMA,
            pltpu.SemaphoreType.DMA,
            pltpu.SemaphoreType.DMA,
            pltpu.SemaphoreType.DMA,
        ],
    )
    return f(widx, eidx, wtab, etab)


def _pad_block(in_ref, out_ref):
    out_ref[...] = jnp.pad(in_ref[...], ((0, 0), (0, PADDED_DIM - WORD_DIM)))


def _pad_table(tab, block_rows):
    rows = tab.shape[0]
    return pl.pallas_call(
        _pad_block,
        grid=(rows // block_rows,),
        in_specs=[pl.BlockSpec((block_rows, WORD_DIM), lambda i: (i, 0))],
        out_specs=pl.BlockSpec((block_rows, PADDED_DIM), lambda i: (i, 0)),
        out_shape=jax.ShapeDtypeStruct((rows, PADDED_DIM), jnp.float32),
    )(tab)


def kernel(words_idx, extwords_idx, word_emb, ext_emb):
    wtab = _pad_table(word_emb, 25000)
    etab = _pad_table(ext_emb, 25000)
    out = _emb_call(words_idx, extwords_idx, wtab, etab)
    return out.reshape(words_idx.shape + (WORD_DIM,))
